# baseline (speedup 1.0000x reference)
import functools

import jax
import jax.numpy as jnp
from jax import lax
from jax.experimental import pallas as pl
from jax.experimental.pallas import tpu as pltpu
from jax.experimental.pallas import tpu_sc as plsc

NUM_SPEAKERS = 1_000_000
EMBED_DIM = 64
BATCH = 16384

NUM_CORES = 2
NUM_SUBCORES = 16
NUM_WORKERS = NUM_CORES * NUM_SUBCORES
B_PER_W = BATCH // NUM_WORKERS
GROUP = 16
NGROUP = B_PER_W // GROUP
NSLOT = 8
BLK = 128

_mesh = plsc.VectorSubcoreMesh(core_axis_name="c", subcore_axis_name="s")


@functools.partial(
    pl.kernel,
    mesh=_mesh,
    out_type=jax.ShapeDtypeStruct((EMBED_DIM * BATCH,), jnp.float32),
    scratch_types=[
        pltpu.VMEM((B_PER_W,), jnp.int32),
        pltpu.VMEM((NSLOT * EMBED_DIM, BLK), jnp.float32),
        pltpu.VMEM((EMBED_DIM, B_PER_W), jnp.float32),
        pltpu.SemaphoreType.DMA,
    ],
    compiler_params=pltpu.CompilerParams(needs_layout_passes=False),
)
def _gather_t(idx_hbm, table_t_hbm, out_t_hbm, idx_v, blk_v, val_v, sem):
    wid = lax.axis_index("s") * NUM_CORES + lax.axis_index("c")
    base = pl.multiple_of(wid * B_PER_W, B_PER_W)
    pltpu.sync_copy(idx_hbm.at[pl.ds(base, B_PER_W)], idx_v)
    lane = lax.iota(jnp.int32, 16)

    def fire(c_scalar, slot):
        col0 = pl.multiple_of(c_scalar * BLK, BLK)
        pltpu.async_copy(
            table_t_hbm.at[:, pl.ds(col0, BLK)],
            blk_v.at[pl.ds(slot * EMBED_DIM, EMBED_DIM), :],
            sem,
        )

    def wait_one(slot):
        pltpu.make_async_copy(
            table_t_hbm.at[:, pl.ds(0, BLK)],
            blk_v.at[pl.ds(slot * EMBED_DIM, EMBED_DIM), :],
            sem,
        ).wait()

    iv0 = idx_v[pl.ds(0, GROUP)]
    cv0 = lax.shift_right_logical(iv0, 7)
    for l in range(NSLOT):
        fire(cv0[l], l)

    def body(g, _):
        k0 = g * GROUP
        iv = idx_v[pl.ds(k0, GROUP)]
        cv = lax.shift_right_logical(iv, 7)
        lv = jnp.bitwise_and(iv, BLK - 1)
        nk0 = jnp.minimum(k0 + GROUP, B_PER_W - GROUP)
        ivn = idx_v[pl.ds(nk0, GROUP)]
        cvn = lax.shift_right_logical(ivn, 7)
        for l in range(GROUP):
            slot = l % NSLOT
            wait_one(slot)
            lam = lv[l]
            col = jnp.full((16,), lam, jnp.int32)
            kk = jnp.full((16,), k0 + l, jnp.int32)
            for j in range(EMBED_DIM // 16):
                rows = slot * EMBED_DIM + j * 16 + lane
                v = plsc.load_gather(blk_v, [rows, col])
                plsc.store_scatter(val_v, [j * 16 + lane, kk], v)
            c2 = cv[l + NSLOT] if l < GROUP - NSLOT else cvn[l - (GROUP - NSLOT)]
            fire(c2, slot)
        return _

    lax.fori_loop(0, NGROUP, body, None)
    for l in range(NSLOT):
        wait_one(l)
    writes = []
    for d in range(EMBED_DIM):
        writes.append(
            pltpu.make_async_copy(
                val_v.at[d],
                out_t_hbm.at[pl.ds(d * BATCH + base, B_PER_W)],
                sem,
            )
        )
        writes[-1].start()
    for w in writes:
        w.wait()


def kernel(i, table):
    idx = i.astype(jnp.int32)
    flat = _gather_t(idx, table.T)
    return flat.reshape(EMBED_DIM, BATCH).T[:, :, None]


# speedup vs baseline: 1.0085x; 1.0085x over previous
import functools

import jax
import jax.numpy as jnp
from jax import lax
from jax.experimental import pallas as pl
from jax.experimental.pallas import tpu as pltpu
from jax.experimental.pallas import tpu_sc as plsc

NUM_SPEAKERS = 1_000_000
EMBED_DIM = 64
BATCH = 16384

NUM_CORES = 2
NUM_SUBCORES = 16
NUM_WORKERS = NUM_CORES * NUM_SUBCORES
B_PER_W = BATCH // NUM_WORKERS
GROUP = 16
NGROUP = B_PER_W // GROUP
NSLOT = 8
BLK = 128

_mesh = plsc.VectorSubcoreMesh(core_axis_name="c", subcore_axis_name="s")


@functools.partial(
    pl.kernel,
    mesh=_mesh,
    out_type=jax.ShapeDtypeStruct((EMBED_DIM, BATCH), jnp.float32),
    scratch_types=[
        pltpu.VMEM((B_PER_W,), jnp.int32),
        pltpu.VMEM((NSLOT * EMBED_DIM, BLK), jnp.float32),
        pltpu.VMEM((EMBED_DIM, B_PER_W), jnp.float32),
        pltpu.SemaphoreType.DMA,
    ],
    compiler_params=pltpu.CompilerParams(needs_layout_passes=False),
)
def _gather_t(idx_hbm, table_t_hbm, out_t_hbm, idx_v, blk_v, val_v, sem):
    wid = lax.axis_index("s") * NUM_CORES + lax.axis_index("c")
    base = pl.multiple_of(wid * B_PER_W, B_PER_W)
    pltpu.sync_copy(idx_hbm.at[pl.ds(base, B_PER_W)], idx_v)
    lane = lax.iota(jnp.int32, 16)

    def fire(c_scalar, slot):
        col0 = pl.multiple_of(c_scalar * BLK, BLK)
        pltpu.async_copy(
            table_t_hbm.at[:, pl.ds(col0, BLK)],
            blk_v.at[pl.ds(slot * EMBED_DIM, EMBED_DIM), :],
            sem,
        )

    def wait_one(slot):
        pltpu.make_async_copy(
            table_t_hbm.at[:, pl.ds(0, BLK)],
            blk_v.at[pl.ds(slot * EMBED_DIM, EMBED_DIM), :],
            sem,
        ).wait()

    iv0 = idx_v[pl.ds(0, GROUP)]
    cv0 = lax.shift_right_logical(iv0, 7)
    for l in range(NSLOT):
        fire(cv0[l], l)

    def body(g, _):
        k0 = g * GROUP
        iv = idx_v[pl.ds(k0, GROUP)]
        cv = lax.shift_right_logical(iv, 7)
        lv = jnp.bitwise_and(iv, BLK - 1)
        nk0 = jnp.minimum(k0 + GROUP, B_PER_W - GROUP)
        ivn = idx_v[pl.ds(nk0, GROUP)]
        cvn = lax.shift_right_logical(ivn, 7)
        for l in range(GROUP):
            slot = l % NSLOT
            wait_one(slot)
            lam = lv[l]
            col = jnp.full((16,), lam, jnp.int32)
            kk = jnp.full((16,), k0 + l, jnp.int32)
            for j in range(EMBED_DIM // 16):
                rows = slot * EMBED_DIM + j * 16 + lane
                v = plsc.load_gather(blk_v, [rows, col])
                plsc.store_scatter(val_v, [j * 16 + lane, kk], v)
            c2 = cv[l + NSLOT] if l < GROUP - NSLOT else cvn[l - (GROUP - NSLOT)]
            fire(c2, slot)
        return _

    lax.fori_loop(0, NGROUP, body, None)
    for l in range(NSLOT):
        wait_one(l)
    pltpu.sync_copy(val_v, out_t_hbm.at[:, pl.ds(base, B_PER_W)])


def kernel(i, table):
    idx = i.astype(jnp.int32)
    out_t = _gather_t(idx, table.T)
    return out_t.T[:, :, None]
